# Initial kernel scaffold; baseline (speedup 1.0000x reference)
import jax
import jax.numpy as jnp
from jax.experimental import pallas as pl


def kernel(x, L_values, L_rows, L_cols, weight, bias):
    raise NotImplementedError("write your pallas kernel here")


# speedup vs baseline: 1.9071x; 1.9071x over previous
import functools

import jax
import jax.numpy as jnp
from jax import lax
from jax.experimental import pallas as pl
from jax.experimental.pallas import tpu as pltpu
from jax.experimental.pallas import tpu_sc as plsc

M = 16384
NNZ = 268435
N = 4
FIN = 64
K = 3
OUT = 64
F = N * FIN
NLANE = 16

NW = 32
ROWS_PER_W = M // NW
SUB = 64
NSUB = ROWS_PER_W // SUB
NBLK = M // SUB
EC = 64
NNZ_PAD = ((NNZ + EC - 1) // EC) * EC
BOUNDS_PAD = NBLK + NLANE

BM = 1024


def _spmm_sc_body(v_hbm, vals_hbm, cols_hbm, rows_hbm, bounds_hbm, out_hbm,
                  bounds_v, rows_v, cols_v, vals_v, gbuf, acc, sem):
    wid = lax.axis_index("s") * 2 + lax.axis_index("c")

    pltpu.sync_copy(bounds_hbm.at[pl.ds(wid * NSUB, NLANE)], bounds_v)
    bvec = bounds_v[...]

    bvals = [bvec[i] for i in range(NSUB + 1)]

    def sub_block(b, carry):
        sub_base = wid * ROWS_PER_W + b * SUB
        s_e = bvals[0]
        e_e = bvals[1]
        for i in range(1, NSUB + 1):
            if i < NSUB:
                s_e = jnp.where(b == i, bvals[i], s_e)
            e_e = jnp.where(b + 1 == i, bvals[i], e_e)
        c0 = s_e // EC
        c1 = (e_e + EC - 1) // EC

        def zero_row(r, carry2):
            z = jnp.zeros((NLANE,), jnp.float32)
            for ff in range(F // NLANE):
                acc[r, pl.ds(ff * NLANE, NLANE)] = z
            return carry2

        lax.fori_loop(0, SUB, zero_row, 0)

        def chunk(c, carry2):
            e0 = c * EC
            pltpu.sync_copy(rows_hbm.at[pl.ds(e0, EC)], rows_v)
            pltpu.sync_copy(cols_hbm.at[pl.ds(e0, EC)], cols_v)
            pltpu.sync_copy(vals_hbm.at[pl.ds(e0, EC)], vals_v)
            pltpu.async_copy(v_hbm.at[cols_v], gbuf, sem).wait()

            def group(g, carry3):
                rvec = rows_v[pl.ds(g * NLANE, NLANE)] - sub_base
                vvec = vals_v[pl.ds(g * NLANE, NLANE)]
                for j in range(NLANE):
                    row_j = rvec[j]
                    val_j = vvec[j]

                    @pl.when((row_j >= 0) & (row_j < SUB))
                    def _():
                        for f in range(F // NLANE):
                            fs = pl.ds(f * NLANE, NLANE)
                            acc[row_j, fs] = acc[row_j, fs] + val_j * gbuf[g * NLANE + j, fs]
                return carry3

            lax.fori_loop(0, EC // NLANE, group, 0)
            return carry2

        lax.fori_loop(c0, c1, chunk, 0)
        pltpu.sync_copy(acc, out_hbm.at[pl.ds(sub_base, SUB)])
        return carry

    lax.fori_loop(0, NSUB, sub_block, 0)


def _spmm_sc(v, vals_p, cols_p, rows_p, bounds):
    mesh = plsc.VectorSubcoreMesh(core_axis_name="c", subcore_axis_name="s")
    fn = pl.kernel(
        _spmm_sc_body,
        mesh=mesh,
        out_type=jax.ShapeDtypeStruct((M, F), jnp.float32),
        scratch_types=[
            pltpu.VMEM((NLANE,), jnp.int32),
            pltpu.VMEM((EC,), jnp.int32),
            pltpu.VMEM((EC,), jnp.int32),
            pltpu.VMEM((EC,), jnp.float32),
            pltpu.VMEM((EC, F), jnp.float32),
            pltpu.VMEM((SUB, F), jnp.float32),
            pltpu.SemaphoreType.DMA,
        ],
    )
    return fn(v, vals_p, cols_p, rows_p, bounds)


def _gemm_body(x0_ref, x1_ref, y2_ref, w_ref, b_ref, o_ref):
    xcat = jnp.concatenate([x0_ref[...], x1_ref[...], y2_ref[...]], axis=1)
    z = lax.dot_general(
        xcat, w_ref[...], (((1,), (0,)), ((), ())),
        preferred_element_type=jnp.float32,
        precision=lax.Precision.HIGHEST,
    ) + b_ref[...]
    o_ref[...] = jnp.where(z > 0, z, jnp.exp(jnp.minimum(z, 0.0)) - 1.0)


def _gemm_elu(x0, x1, y2, wbd, bias_t):
    return pl.pallas_call(
        _gemm_body,
        grid=(M // BM,),
        in_specs=[
            pl.BlockSpec((BM, F), lambda i: (i, 0)),
            pl.BlockSpec((BM, F), lambda i: (i, 0)),
            pl.BlockSpec((BM, F), lambda i: (i, 0)),
            pl.BlockSpec((3 * F, N * OUT), lambda i: (0, 0)),
            pl.BlockSpec((1, N * OUT), lambda i: (0, 0)),
        ],
        out_specs=pl.BlockSpec((BM, N * OUT), lambda i: (i, 0)),
        out_shape=jax.ShapeDtypeStruct((M, N * OUT), jnp.float32),
    )(x0, x1, y2, wbd, bias_t)


def kernel(x, L_values, L_rows, L_cols, weight, bias):
    rows = L_rows.astype(jnp.int32)
    cols = L_cols.astype(jnp.int32)

    x0 = jnp.transpose(x, (1, 0, 2)).reshape(M, F)

    pad = NNZ_PAD - NNZ
    rows_p = jnp.concatenate([rows, jnp.full((pad,), M, jnp.int32)])
    cols_p = jnp.concatenate([cols, jnp.zeros((pad,), jnp.int32)])
    vals_p = jnp.concatenate([L_values, jnp.zeros((pad,), jnp.float32)])

    bounds = jnp.searchsorted(rows, jnp.arange(NBLK + 1, dtype=jnp.int32) * SUB).astype(jnp.int32)
    bounds = jnp.concatenate([bounds, jnp.full((BOUNDS_PAD - NBLK - 1,), NNZ, jnp.int32)])

    x1 = _spmm_sc(x0, vals_p, cols_p, rows_p, bounds)
    y2 = _spmm_sc(x1, vals_p, cols_p, rows_p, bounds)

    w = weight.reshape(FIN, K, OUT)
    eye_n = jnp.eye(N, dtype=jnp.float32)

    def bd(wk):
        return (eye_n[:, None, :, None] * wk[None, :, None, :]).reshape(N * FIN, N * OUT)

    wbd = jnp.concatenate([bd(w[:, 0] - w[:, 2]), bd(w[:, 1]), bd(2.0 * w[:, 2])], axis=0)
    bias_t = jnp.tile(bias, (N,)).reshape(1, N * OUT)

    out = _gemm_elu(x0, x1, y2, wbd, bias_t)
    return out.reshape(M, N, OUT).transpose(1, 0, 2)
